# Initial kernel scaffold; baseline (speedup 1.0000x reference)
import jax
import jax.numpy as jnp
from jax.experimental import pallas as pl


def kernel(x, edge_index, batch, Wt, bt, bn0_g, bn0_b, W1, W2, bng, bnb):
    raise NotImplementedError("write your pallas kernel here")


# speedup vs baseline: 2.6226x; 2.6226x over previous
import functools

import jax
import jax.numpy as jnp
from jax import lax
from jax.experimental import pallas as pl
from jax.experimental.pallas import tpu as pltpu
from jax.experimental.pallas import tpu_sc as plsc

N = 10000
D = 128
NLAYER = 3
BN_EPS = 1e-5

NP = 10112
E = 320000
NW = 32
CB = 128
NCHUNK = 80
EPW = NCHUNK * CB
EPAD = NW * EPW
RPT = NP // 16


def _sc_agg_body(h_hbm, src_hbm, dst_hbm, zero_hbm, out_hbm,
                 src_v, dst_v, rows_v, acc_sh, sem):
    c = lax.axis_index("c")
    s = lax.axis_index("s")
    wid = c * 16 + s
    pltpu.sync_copy(src_hbm.at[pl.ds(wid * EPW, EPW)], src_v)
    pltpu.sync_copy(dst_hbm.at[pl.ds(wid * NCHUNK, NCHUNK)], dst_v)
    pltpu.sync_copy(zero_hbm, rows_v)
    base = s * RPT
    for k in range(4):
        pltpu.sync_copy(rows_v, acc_sh.at[pl.ds(base + k * CB, CB)])
    pltpu.sync_copy(rows_v.at[pl.ds(0, RPT - 4 * CB)],
                    acc_sh.at[pl.ds(base + 4 * CB, RPT - 4 * CB)])
    plsc.subcore_barrier()

    def body(j, carry):
        pltpu.async_copy(h_hbm.at[src_v.at[pl.ds(j * CB, CB)]], rows_v,
                         sem).wait()
        pltpu.sync_copy(rows_v, acc_sh.at[dst_v.at[j]], add=True)
        return carry

    lax.fori_loop(0, NCHUNK, body, 0)
    plsc.subcore_barrier()
    pltpu.sync_copy(acc_sh.at[pl.ds(base, RPT)], out_hbm.at[c, pl.ds(base, RPT)])


@functools.cache
def _get_sc_agg():
    return functools.partial(
        pl.kernel,
        out_type=jax.ShapeDtypeStruct((2, NP, D), jnp.float32),
        mesh=plsc.VectorSubcoreMesh(core_axis_name="c", subcore_axis_name="s"),
        scratch_types=[
            pltpu.VMEM((EPW,), jnp.int32),
            pltpu.VMEM((NCHUNK, CB), jnp.int32),
            pltpu.VMEM((CB, D), jnp.float32),
            pltpu.VMEM_SHARED((NP, D), jnp.float32),
            pltpu.SemaphoreType.DMA,
        ],
    )(_sc_agg_body)


def _tc_transform_body(x_ref, wt_ref, bt_ref, g_ref, b_ref, o_ref):
    h = jnp.dot(x_ref[...], wt_ref[...], preferred_element_type=jnp.float32)
    h = h + bt_ref[...]
    mu = jnp.mean(h, axis=0, keepdims=True)
    dvar = h - mu
    var = jnp.mean(dvar * dvar, axis=0, keepdims=True)
    o_ref[...] = dvar * lax.rsqrt(var + BN_EPS) * g_ref[...] + b_ref[...]


def _tc_layer_body(h_ref, p_ref, w1_ref, w2_ref, g_ref, b_ref, o_ref):
    agg = p_ref[0, :N, :] + p_ref[1, :N, :]
    m = h_ref[...] + agg
    m = jnp.maximum(jnp.dot(m, w1_ref[...], preferred_element_type=jnp.float32), 0.0)
    m = jnp.maximum(jnp.dot(m, w2_ref[...], preferred_element_type=jnp.float32), 0.0)
    mu = jnp.mean(m, axis=0, keepdims=True)
    dvar = m - mu
    var = jnp.mean(dvar * dvar, axis=0, keepdims=True)
    o_ref[...] = dvar * lax.rsqrt(var + BN_EPS) * g_ref[...] + b_ref[...]


_f32 = jnp.float32


def kernel(x, edge_index, batch, Wt, bt, bn0_g, bn0_b, W1, W2, bng, bnb):
    del batch
    npad = EPAD - E
    src = jnp.concatenate([edge_index[0], jnp.zeros((npad,), jnp.int32)])
    dst = jnp.concatenate([edge_index[1], jnp.full((npad,), N, jnp.int32)])
    dst2d = dst.reshape(NW * NCHUNK, CB)
    zero_blk = jnp.zeros((CB, D), _f32)

    transform = pl.pallas_call(
        _tc_transform_body,
        out_shape=jax.ShapeDtypeStruct((N, D), _f32),
    )
    h = transform(x, Wt, bt.reshape(1, D), bn0_g.reshape(1, D),
                  bn0_b.reshape(1, D))

    layer = pl.pallas_call(
        _tc_layer_body,
        out_shape=jax.ShapeDtypeStruct((N, D), _f32),
    )
    sc_agg = _get_sc_agg()
    for i in range(NLAYER):
        parts = sc_agg(h, src, dst2d, zero_blk)
        h = layer(h, parts, W1[i], W2[i], bng[i].reshape(1, D),
                  bnb[i].reshape(1, D))
    return h
